# Initial kernel scaffold; baseline (speedup 1.0000x reference)
import jax
import jax.numpy as jnp
from jax.experimental import pallas as pl


def kernel(node_rep, edge_rep, edge_index, node_W1, node_g1, node_b1, node_W2, node_g2, node_b2, edge_W1, edge_g1, edge_b1, edge_W2, edge_g2, edge_b2):
    raise NotImplementedError("write your pallas kernel here")


# speedup vs baseline: 2.3248x; 2.3248x over previous
import functools

import jax
import jax.numpy as jnp
from jax import lax
from jax.experimental import pallas as pl
from jax.experimental.pallas import tpu as pltpu
from jax.experimental.pallas import tpu_sc as plsc

_N = 10000
_E = 160000
_H = 128
_R2E = 2 * _E
_CH = 80
_NT = 16
_NP = 10240
_NPT = _NP // _NT
_EPS = 1e-5

_f32 = jnp.float32


def _sc_mesh():
    return plsc.VectorSubcoreMesh(core_axis_name="c", subcore_axis_name="s")


def _k1_body(era, idx2_h, zz,
             se_o, t_o,
             idx_v, rows_v, tab_sh):
    c = lax.axis_index("c")
    s = lax.axis_index("s")
    pltpu.sync_copy(zz, tab_sh.at[pl.ds(s * _NPT, _NPT)])
    plsc.subcore_barrier()

    rpt = _R2E // _NT
    nch = rpt // _CH
    ibase = c * _R2E

    def step(j, carry):
        r0 = s * rpt + j * _CH
        pltpu.sync_copy(idx2_h.at[pl.ds(ibase + r0, _CH)], idx_v)
        pltpu.sync_copy(era.at[pl.ds(r0, _CH)], rows_v)
        pltpu.sync_copy(rows_v, tab_sh.at[idx_v], add=True)
        return carry

    lax.fori_loop(0, nch, step, 0)
    plsc.subcore_barrier()

    @pl.when(c == 0)
    def _():
        pltpu.sync_copy(tab_sh.at[pl.ds(s * _NPT, _NPT)],
                        se_o.at[pl.ds(s * _NPT, _NPT)])

    @pl.when(c == 1)
    def _():
        pltpu.sync_copy(tab_sh.at[pl.ds(s * _NPT, _NPT)],
                        t_o.at[pl.ds(s * _NPT, _NPT)])


def _run_k1(edge_rep, idx2):
    zz = jnp.zeros((_NPT, _H), _f32)
    fn = pl.kernel(
        _k1_body,
        out_type=(jax.ShapeDtypeStruct((_NP, _H), _f32),
                  jax.ShapeDtypeStruct((_NP, _H), _f32)),
        mesh=_sc_mesh(),
        scratch_types=[
            pltpu.VMEM((_CH,), jnp.int32),
            pltpu.VMEM((_CH, _H), _f32),
            pltpu.VMEM_SHARED((_NP, _H), _f32),
        ],
    )
    return fn(edge_rep, idx2, zz)


def _kd_body(atoms_h, zz, oo,
             d0_o, d1_o,
             idx_v, ones_v, tab_sh):
    c = lax.axis_index("c")
    s = lax.axis_index("s")
    pltpu.sync_copy(zz, tab_sh.at[pl.ds(s * _NPT, _NPT)])
    pltpu.sync_copy(oo, ones_v)
    plsc.subcore_barrier()

    rpc = _R2E // 2
    rpt = rpc // _NT
    nch = rpt // _CH

    def step(j, carry):
        r0 = c * rpc + s * rpt + j * _CH
        pltpu.sync_copy(atoms_h.at[pl.ds(r0, _CH)], idx_v)
        pltpu.sync_copy(ones_v, tab_sh.at[idx_v], add=True)
        return carry

    lax.fori_loop(0, nch, step, 0)
    plsc.subcore_barrier()

    @pl.when(c == 0)
    def _():
        pltpu.sync_copy(tab_sh.at[pl.ds(s * _NPT, _NPT)],
                        d0_o.at[pl.ds(s * _NPT, _NPT)])

    @pl.when(c == 1)
    def _():
        pltpu.sync_copy(tab_sh.at[pl.ds(s * _NPT, _NPT)],
                        d1_o.at[pl.ds(s * _NPT, _NPT)])


def _run_kd(atoms):
    zz = jnp.zeros((_NPT, _H), _f32)
    oo = jnp.ones((_CH, _H), _f32)
    fn = pl.kernel(
        _kd_body,
        out_type=(jax.ShapeDtypeStruct((_NP, _H), _f32),
                  jax.ShapeDtypeStruct((_NP, _H), _f32)),
        mesh=_sc_mesh(),
        scratch_types=[
            pltpu.VMEM((_CH,), jnp.int32),
            pltpu.VMEM((_CH, _H), _f32),
            pltpu.VMEM_SHARED((_NP, _H), _f32),
        ],
    )
    return fn(atoms, zz, oo)


def _k2_body(tab2_h, atoms_h, aoth_h, zz,
             u_o, v_o,
             idxd_v, idxs_v, rows_v, tab_sh, sem):
    c = lax.axis_index("c")
    s = lax.axis_index("s")
    pltpu.sync_copy(zz, tab_sh.at[pl.ds(s * _NPT, _NPT)])
    plsc.subcore_barrier()

    rpt = _R2E // _NT
    nch = rpt // _CH
    off = c * _NP

    def step(j, carry):
        r0 = s * rpt + j * _CH
        pltpu.sync_copy(atoms_h.at[pl.ds(r0, _CH)], idxd_v)
        pltpu.sync_copy(aoth_h.at[pl.ds(r0, _CH)], idxs_v)
        for k in range(_CH // 16):
            sl = pl.ds(k * 16, 16)
            idxs_v[sl] = idxs_v[sl] + off
        pltpu.async_copy(tab2_h.at[idxs_v], rows_v, sem).wait()
        pltpu.sync_copy(rows_v, tab_sh.at[idxd_v], add=True)
        return carry

    lax.fori_loop(0, nch, step, 0)
    plsc.subcore_barrier()

    @pl.when(c == 0)
    def _():
        pltpu.sync_copy(tab_sh.at[pl.ds(s * _NPT, _NPT)],
                        u_o.at[pl.ds(s * _NPT, _NPT)])

    @pl.when(c == 1)
    def _():
        pltpu.sync_copy(tab_sh.at[pl.ds(s * _NPT, _NPT)],
                        v_o.at[pl.ds(s * _NPT, _NPT)])


def _run_k2(tab2, atoms, aoth):
    zz = jnp.zeros((_NPT, _H), _f32)
    fn = pl.kernel(
        _k2_body,
        out_type=(jax.ShapeDtypeStruct((_NP, _H), _f32),
                  jax.ShapeDtypeStruct((_NP, _H), _f32)),
        mesh=_sc_mesh(),
        scratch_types=[
            pltpu.VMEM((_CH,), jnp.int32),
            pltpu.VMEM((_CH,), jnp.int32),
            pltpu.VMEM((_CH, _H), _f32),
            pltpu.VMEM_SHARED((_NP, _H), _f32),
            pltpu.SemaphoreType.DMA,
        ],
    )
    return fn(tab2, atoms, aoth, zz)


def _k3_body(yal_h, yah_h, ybl_h, ybh_h, atoms_h, aoth_h, iota_h, g_o,
             idxa_v, idxb_v, buf_v, iota_v, stage_sh, sem):
    c = lax.axis_index("c")
    s = lax.axis_index("s")
    wid = s * 2 + c
    rpw = _R2E // 32
    nch = rpw // _CH
    pltpu.sync_copy(iota_h.at[pl.ds(s * _CH, _CH)], iota_v)

    def step(j, carry):
        r0 = wid * rpw + j * _CH
        pltpu.sync_copy(atoms_h.at[pl.ds(r0, _CH)], idxa_v)
        pltpu.sync_copy(aoth_h.at[pl.ds(r0, _CH)], idxb_v)
        for half, (ya_h, yb_h) in enumerate(((yal_h, ybl_h),
                                             (yah_h, ybh_h))):
            pltpu.async_copy(ya_h.at[idxa_v], buf_v, sem).wait()
            pltpu.sync_copy(buf_v, stage_sh.at[pl.ds(s * _CH, _CH)])
            pltpu.async_copy(yb_h.at[idxb_v], buf_v, sem).wait()
            pltpu.sync_copy(buf_v, stage_sh.at[iota_v], add=True)
            pltpu.sync_copy(stage_sh.at[pl.ds(s * _CH, _CH)],
                            g_o.at[pl.ds(r0, _CH), pl.ds(half * _H, _H)])
        return carry

    lax.fori_loop(0, nch, step, 0)


def _run_k3(ya, yb, atoms, aoth):
    iota = jnp.arange(_NT * _CH, dtype=jnp.int32)
    fn = pl.kernel(
        _k3_body,
        out_type=jax.ShapeDtypeStruct((_R2E, 2 * _H), _f32),
        mesh=_sc_mesh(),
        scratch_types=[
            pltpu.VMEM((_CH,), jnp.int32),
            pltpu.VMEM((_CH,), jnp.int32),
            pltpu.VMEM((_CH, _H), _f32),
            pltpu.VMEM((_CH,), jnp.int32),
            pltpu.VMEM_SHARED((_NT * _CH, _H), _f32),
            pltpu.SemaphoreType.DMA,
        ],
    )
    return fn(ya[:, :_H], ya[:, _H:], yb[:, :_H], yb[:, _H:],
              atoms, aoth, iota)


def _dot(a, b):
    return lax.dot_general(a, b, (((1,), (0,)), ((), ())),
                           preferred_element_type=_f32,
                           precision=lax.Precision.HIGHEST)


def _wsplit(w):
    return [w[i * _H:(i + 1) * _H] for i in range(7)]


def _feat_body(se_ref, t_ref, u_ref, v_ref, nr_ref, d0_ref, d1_ref, ew1_ref,
               ya_ref, yb_ref, xn_ref):
    w = _wsplit(ew1_ref[...])
    b_m = w[1] + w[2] + w[3] + w[4]
    c_m = w[1] + w[3]
    d_m = w[1] + w[2]
    f_m = w[5] + w[6]
    g_m = w[5]
    se = se_ref[...]
    t = t_ref[...]
    nr = nr_ref[...]
    d = d0_ref[...][:, 0:1] + d1_ref[...][:, 0:1]
    d2 = d * d
    ya_ref[...] = _dot(d * se, b_m) + _dot(t, c_m) + _dot(d * nr, f_m)
    yb_ref[...] = _dot(se, d_m) + _dot(nr, g_m)
    xn_ref[...] = jnp.concatenate(
        [nr, d2 * se, d * t, u_ref[...], se, d2 * nr, v_ref[...]], axis=1)


def _run_feat(se, t, u, v, nr, d0, d1, ew1):
    nb, blk = 10, _N // 10
    grid = (nb,)
    io128 = pl.BlockSpec((blk, _H), lambda i: (i, 0))
    return pl.pallas_call(
        _feat_body,
        grid=grid,
        in_specs=[io128, io128, io128, io128, io128, io128, io128,
                  pl.BlockSpec((7 * _H, 2 * _H), lambda i: (0, 0))],
        out_specs=[pl.BlockSpec((blk, 2 * _H), lambda i: (i, 0)),
                   pl.BlockSpec((blk, 2 * _H), lambda i: (i, 0)),
                   pl.BlockSpec((blk, 7 * _H), lambda i: (i, 0))],
        out_shape=[jax.ShapeDtypeStruct((_N, 2 * _H), _f32),
                   jax.ShapeDtypeStruct((_N, 2 * _H), _f32),
                   jax.ShapeDtypeStruct((_N, 7 * _H), _f32)],
    )(se, t, u, v, nr, d0, d1, ew1)


def _bn_coeffs(stats_ref, g_ref, b_ref, n):
    sums = stats_ref[0:1, :]
    sqs = stats_ref[1:2, :]
    mean = sums * (1.0 / n)
    var = sqs * (1.0 / n) - mean * mean
    scale = g_ref[...] * lax.rsqrt(var + _EPS)
    shift = b_ref[...] - mean * scale
    return scale, shift


def _ep1_body(er_ref, g_ref, ew1_ref, stats_ref, acc_ref):
    i = pl.program_id(0)

    @pl.when(i == 0)
    def _():
        acc_ref[...] = jnp.zeros_like(acc_ref)

    w = ew1_ref[...]
    a_m = w[0:_H] + w[_H:2 * _H]
    h = _dot(er_ref[...], a_m) + g_ref[...]
    acc_ref[0:1, :] += jnp.sum(h, axis=0, keepdims=True)
    acc_ref[1:2, :] += jnp.sum(h * h, axis=0, keepdims=True)
    stats_ref[...] = acc_ref[...]


def _run_ep1(er, g, ew1, blk=2000):
    nb = _R2E // blk
    return pl.pallas_call(
        _ep1_body,
        grid=(nb,),
        in_specs=[pl.BlockSpec((blk, _H), lambda i: (i, 0)),
                  pl.BlockSpec((blk, 2 * _H), lambda i: (i, 0)),
                  pl.BlockSpec((7 * _H, 2 * _H), lambda i: (0, 0))],
        out_specs=pl.BlockSpec((8, 2 * _H), lambda i: (0, 0)),
        out_shape=jax.ShapeDtypeStruct((8, 2 * _H), _f32),
        scratch_shapes=[pltpu.VMEM((8, 2 * _H), _f32)],
    )(er, g, ew1)


def _ep2_body(er_ref, g_ref, ew1_ref, st1_ref, g1_ref, b1_ref, w2_ref,
              h2_ref, stats_ref, acc_ref):
    i = pl.program_id(0)

    @pl.when(i == 0)
    def _():
        acc_ref[...] = jnp.zeros_like(acc_ref)

    w = ew1_ref[...]
    a_m = w[0:_H] + w[_H:2 * _H]
    h = _dot(er_ref[...], a_m) + g_ref[...]
    scale, shift = _bn_coeffs(st1_ref, g1_ref, b1_ref, float(_R2E))
    r = jnp.maximum(h * scale + shift, 0.0)
    h2 = _dot(r, w2_ref[...])
    acc_ref[0:1, :] += jnp.sum(h2, axis=0, keepdims=True)
    acc_ref[1:2, :] += jnp.sum(h2 * h2, axis=0, keepdims=True)
    h2_ref[...] = h2
    stats_ref[...] = acc_ref[...]


def _run_ep2(er, g, ew1, st1, g1, b1, w2, blk=2000):
    nb = _R2E // blk
    return pl.pallas_call(
        _ep2_body,
        grid=(nb,),
        in_specs=[pl.BlockSpec((blk, _H), lambda i: (i, 0)),
                  pl.BlockSpec((blk, 2 * _H), lambda i: (i, 0)),
                  pl.BlockSpec((7 * _H, 2 * _H), lambda i: (0, 0)),
                  pl.BlockSpec((8, 2 * _H), lambda i: (0, 0)),
                  pl.BlockSpec((1, 2 * _H), lambda i: (0, 0)),
                  pl.BlockSpec((1, 2 * _H), lambda i: (0, 0)),
                  pl.BlockSpec((2 * _H, _H), lambda i: (0, 0))],
        out_specs=[pl.BlockSpec((blk, _H), lambda i: (i, 0)),
                   pl.BlockSpec((8, _H), lambda i: (0, 0))],
        out_shape=[jax.ShapeDtypeStruct((_R2E, _H), _f32),
                   jax.ShapeDtypeStruct((8, _H), _f32)],
        scratch_shapes=[pltpu.VMEM((8, _H), _f32)],
    )(er, g, ew1, st1, g1, b1, w2)


def _run_p3(h2, st2, g2, b2, n_rows, blk):
    nb = h2.shape[0] // blk
    n = float(n_rows)

    def _p3_body(h2_ref, st2_ref, g2_ref, b2_ref, out_ref):
        scale, shift = _bn_coeffs(st2_ref, g2_ref, b2_ref, n)
        out_ref[...] = jnp.maximum(h2_ref[...] * scale + shift, 0.0)

    return pl.pallas_call(
        _p3_body,
        grid=(nb,),
        in_specs=[pl.BlockSpec((blk, _H), lambda i: (i, 0)),
                  pl.BlockSpec((8, _H), lambda i: (0, 0)),
                  pl.BlockSpec((1, _H), lambda i: (0, 0)),
                  pl.BlockSpec((1, _H), lambda i: (0, 0))],
        out_specs=pl.BlockSpec((blk, _H), lambda i: (i, 0)),
        out_shape=jax.ShapeDtypeStruct((h2.shape[0], _H), _f32),
    )(h2, st2, g2, b2)


def _np1_body(xn_ref, nw1_ref, h1_ref, stats_ref, acc_ref):
    i = pl.program_id(0)

    @pl.when(i == 0)
    def _():
        acc_ref[...] = jnp.zeros_like(acc_ref)

    w = _wsplit(nw1_ref[...])
    wcat = jnp.concatenate(
        [w[0], w[1] + w[2] + w[3] + w[4], w[1] + w[3], w[1] + w[2],
         w[1], w[5] + w[6], w[5]], axis=0)
    h = _dot(xn_ref[...], wcat)
    acc_ref[0:1, :] += jnp.sum(h, axis=0, keepdims=True)
    acc_ref[1:2, :] += jnp.sum(h * h, axis=0, keepdims=True)
    h1_ref[...] = h
    stats_ref[...] = acc_ref[...]


def _run_np1(xn, nw1):
    nb, blk = 10, _N // 10
    return pl.pallas_call(
        _np1_body,
        grid=(nb,),
        in_specs=[pl.BlockSpec((blk, 7 * _H), lambda i: (i, 0)),
                  pl.BlockSpec((7 * _H, 2 * _H), lambda i: (0, 0))],
        out_specs=[pl.BlockSpec((blk, 2 * _H), lambda i: (i, 0)),
                   pl.BlockSpec((8, 2 * _H), lambda i: (0, 0))],
        out_shape=[jax.ShapeDtypeStruct((_N, 2 * _H), _f32),
                   jax.ShapeDtypeStruct((8, 2 * _H), _f32)],
        scratch_shapes=[pltpu.VMEM((8, 2 * _H), _f32)],
    )(xn, nw1)


def _np2_body(h1_ref, st1_ref, g1_ref, b1_ref, w2_ref,
              h2_ref, stats_ref, acc_ref):
    i = pl.program_id(0)

    @pl.when(i == 0)
    def _():
        acc_ref[...] = jnp.zeros_like(acc_ref)

    scale, shift = _bn_coeffs(st1_ref, g1_ref, b1_ref, float(_N))
    r = jnp.maximum(h1_ref[...] * scale + shift, 0.0)
    h2 = _dot(r, w2_ref[...])
    acc_ref[0:1, :] += jnp.sum(h2, axis=0, keepdims=True)
    acc_ref[1:2, :] += jnp.sum(h2 * h2, axis=0, keepdims=True)
    h2_ref[...] = h2
    stats_ref[...] = acc_ref[...]


def _run_np2(h1, st1, g1, b1, w2):
    nb, blk = 10, _N // 10
    return pl.pallas_call(
        _np2_body,
        grid=(nb,),
        in_specs=[pl.BlockSpec((blk, 2 * _H), lambda i: (i, 0)),
                  pl.BlockSpec((8, 2 * _H), lambda i: (0, 0)),
                  pl.BlockSpec((1, 2 * _H), lambda i: (0, 0)),
                  pl.BlockSpec((1, 2 * _H), lambda i: (0, 0)),
                  pl.BlockSpec((2 * _H, _H), lambda i: (0, 0))],
        out_specs=[pl.BlockSpec((blk, _H), lambda i: (i, 0)),
                   pl.BlockSpec((8, _H), lambda i: (0, 0))],
        out_shape=[jax.ShapeDtypeStruct((_N, _H), _f32),
                   jax.ShapeDtypeStruct((8, _H), _f32)],
        scratch_shapes=[pltpu.VMEM((8, _H), _f32)],
    )(h1, st1, g1, b1, w2)


def kernel(node_rep, edge_rep, edge_index,
           node_W1, node_g1, node_b1, node_W2, node_g2, node_b2,
           edge_W1, edge_g1, edge_b1, edge_W2, edge_g2, edge_b2):
    u, v = edge_index[0], edge_index[1]
    atoms = jnp.stack([u, v], axis=1).reshape(-1).astype(jnp.int32)
    aoth = jnp.stack([v, u], axis=1).reshape(-1).astype(jnp.int32)

    idx2 = jnp.concatenate([atoms, aoth])
    se, t = _run_k1(edge_rep, idx2)
    d0, d1 = _run_kd(atoms)
    tab2 = jnp.concatenate(
        [se, node_rep, jnp.zeros((_NP - _N, _H), _f32)], axis=0)
    uu, vv = _run_k2(tab2, atoms, aoth)
    ya, yb, xn = _run_feat(se[:_N], t[:_N], uu[:_N], vv[:_N],
                           node_rep, d0[:_N], d1[:_N], edge_W1)
    g = _run_k3(ya, yb, atoms, aoth)

    eg1 = edge_g1.reshape(1, -1)
    eb1 = edge_b1.reshape(1, -1)
    st1 = _run_ep1(edge_rep, g, edge_W1)
    h2e, st2 = _run_ep2(edge_rep, g, edge_W1, st1, eg1, eb1, edge_W2)
    edge_out = _run_p3(h2e, st2, edge_g2.reshape(1, -1),
                       edge_b2.reshape(1, -1), _R2E, 2000)

    nst1_h1 = _run_np1(xn, node_W1)
    h1n, nst1 = nst1_h1
    h2n, nst2 = _run_np2(h1n, nst1, node_g1.reshape(1, -1),
                         node_b1.reshape(1, -1), node_W2)
    node_out = _run_p3(h2n, nst2, node_g2.reshape(1, -1),
                       node_b2.reshape(1, -1), _N, 1000)

    return (node_out, edge_out)


# speedup vs baseline: 2.7410x; 1.1790x over previous
import functools

import jax
import jax.numpy as jnp
from jax import lax
from jax.experimental import pallas as pl
from jax.experimental.pallas import tpu as pltpu
from jax.experimental.pallas import tpu_sc as plsc

_N = 10000
_E = 160000
_H = 128
_R2E = 2 * _E
_CH = 80
_NT = 16
_NP = 10240
_NPT = _NP // _NT
_EPS = 1e-5

_f32 = jnp.float32


def _sc_mesh():
    return plsc.VectorSubcoreMesh(core_axis_name="c", subcore_axis_name="s")


def _k1_body(era, idx2_h, zz,
             se_o, t_o,
             idx_v, rows_v, tab_sh):
    c = lax.axis_index("c")
    s = lax.axis_index("s")
    pltpu.sync_copy(zz, tab_sh.at[pl.ds(s * _NPT, _NPT)])
    plsc.subcore_barrier()

    rpt = _R2E // _NT
    nch = rpt // _CH
    ibase = c * _R2E

    def step(j, carry):
        r0 = s * rpt + j * _CH
        pltpu.sync_copy(idx2_h.at[pl.ds(ibase + r0, _CH)], idx_v)
        pltpu.sync_copy(era.at[pl.ds(r0, _CH)], rows_v)
        pltpu.sync_copy(rows_v, tab_sh.at[idx_v], add=True)
        return carry

    lax.fori_loop(0, nch, step, 0)
    plsc.subcore_barrier()

    @pl.when(c == 0)
    def _():
        pltpu.sync_copy(tab_sh.at[pl.ds(s * _NPT, _NPT)],
                        se_o.at[pl.ds(s * _NPT, _NPT)])

    @pl.when(c == 1)
    def _():
        pltpu.sync_copy(tab_sh.at[pl.ds(s * _NPT, _NPT)],
                        t_o.at[pl.ds(s * _NPT, _NPT)])


def _run_k1(edge_rep, idx2):
    zz = jnp.zeros((_NPT, _H), _f32)
    fn = pl.kernel(
        _k1_body,
        out_type=(jax.ShapeDtypeStruct((_NP, _H), _f32),
                  jax.ShapeDtypeStruct((_NP, _H), _f32)),
        mesh=_sc_mesh(),
        scratch_types=[
            pltpu.VMEM((_CH,), jnp.int32),
            pltpu.VMEM((_CH, _H), _f32),
            pltpu.VMEM_SHARED((_NP, _H), _f32),
        ],
    )
    return fn(edge_rep, idx2, zz)


def _kd_body(atoms_h, zz, oo,
             d0_o, d1_o,
             idx_v, ones_v, tab_sh):
    c = lax.axis_index("c")
    s = lax.axis_index("s")
    pltpu.sync_copy(zz, tab_sh.at[pl.ds(s * _NPT, _NPT)])
    pltpu.sync_copy(oo, ones_v)
    plsc.subcore_barrier()

    rpc = _R2E // 2
    rpt = rpc // _NT
    nch = rpt // _CH

    def step(j, carry):
        r0 = c * rpc + s * rpt + j * _CH
        pltpu.sync_copy(atoms_h.at[pl.ds(r0, _CH)], idx_v)
        pltpu.sync_copy(ones_v, tab_sh.at[idx_v], add=True)
        return carry

    lax.fori_loop(0, nch, step, 0)
    plsc.subcore_barrier()

    @pl.when(c == 0)
    def _():
        pltpu.sync_copy(tab_sh.at[pl.ds(s * _NPT, _NPT)],
                        d0_o.at[pl.ds(s * _NPT, _NPT)])

    @pl.when(c == 1)
    def _():
        pltpu.sync_copy(tab_sh.at[pl.ds(s * _NPT, _NPT)],
                        d1_o.at[pl.ds(s * _NPT, _NPT)])


def _run_kd(atoms):
    zz = jnp.zeros((_NPT, _H), _f32)
    oo = jnp.ones((_CH, _H), _f32)
    fn = pl.kernel(
        _kd_body,
        out_type=(jax.ShapeDtypeStruct((_NP, _H), _f32),
                  jax.ShapeDtypeStruct((_NP, _H), _f32)),
        mesh=_sc_mesh(),
        scratch_types=[
            pltpu.VMEM((_CH,), jnp.int32),
            pltpu.VMEM((_CH, _H), _f32),
            pltpu.VMEM_SHARED((_NP, _H), _f32),
        ],
    )
    return fn(atoms, zz, oo)


def _k2_body(tab2_h, atoms_h, aoth_h, zz,
             u_o, v_o,
             idxd_v, idxs_v, rows_v, tab_sh, sem):
    c = lax.axis_index("c")
    s = lax.axis_index("s")
    pltpu.sync_copy(zz, tab_sh.at[pl.ds(s * _NPT, _NPT)])
    plsc.subcore_barrier()

    rpt = _R2E // _NT
    nch = rpt // _CH
    off = c * _NP

    def step(j, carry):
        r0 = s * rpt + j * _CH
        pltpu.sync_copy(atoms_h.at[pl.ds(r0, _CH)], idxd_v)
        pltpu.sync_copy(aoth_h.at[pl.ds(r0, _CH)], idxs_v)
        for k in range(_CH // 16):
            sl = pl.ds(k * 16, 16)
            idxs_v[sl] = idxs_v[sl] + off
        pltpu.async_copy(tab2_h.at[idxs_v], rows_v, sem).wait()
        pltpu.sync_copy(rows_v, tab_sh.at[idxd_v], add=True)
        return carry

    lax.fori_loop(0, nch, step, 0)
    plsc.subcore_barrier()

    @pl.when(c == 0)
    def _():
        pltpu.sync_copy(tab_sh.at[pl.ds(s * _NPT, _NPT)],
                        u_o.at[pl.ds(s * _NPT, _NPT)])

    @pl.when(c == 1)
    def _():
        pltpu.sync_copy(tab_sh.at[pl.ds(s * _NPT, _NPT)],
                        v_o.at[pl.ds(s * _NPT, _NPT)])


def _run_k2(tab2, atoms, aoth):
    zz = jnp.zeros((_NPT, _H), _f32)
    fn = pl.kernel(
        _k2_body,
        out_type=(jax.ShapeDtypeStruct((_NP, _H), _f32),
                  jax.ShapeDtypeStruct((_NP, _H), _f32)),
        mesh=_sc_mesh(),
        scratch_types=[
            pltpu.VMEM((_CH,), jnp.int32),
            pltpu.VMEM((_CH,), jnp.int32),
            pltpu.VMEM((_CH, _H), _f32),
            pltpu.VMEM_SHARED((_NP, _H), _f32),
            pltpu.SemaphoreType.DMA,
        ],
    )
    return fn(tab2, atoms, aoth, zz)


_bf16 = jnp.bfloat16


def _k3_body(ya_h, yb_h, atoms_h, aoth_h, ga_o, gb_o,
             idxa_v, idxb_v, bufa_v, bufb_v, sema, semb):
    c = lax.axis_index("c")
    s = lax.axis_index("s")
    wid = s * 2 + c
    rpw = _R2E // 32
    nch = rpw // _CH

    def step(j, carry):
        r0 = wid * rpw + j * _CH
        pltpu.sync_copy(atoms_h.at[pl.ds(r0, _CH)], idxa_v)
        pltpu.sync_copy(aoth_h.at[pl.ds(r0, _CH)], idxb_v)
        ca = pltpu.async_copy(ya_h.at[idxa_v], bufa_v, sema)
        cb = pltpu.async_copy(yb_h.at[idxb_v], bufb_v, semb)
        ca.wait()
        pltpu.sync_copy(bufa_v, ga_o.at[pl.ds(r0, _CH)])
        cb.wait()
        pltpu.sync_copy(bufb_v, gb_o.at[pl.ds(r0, _CH)])
        return carry

    lax.fori_loop(0, nch, step, 0)


def _run_k3(ya, yb, atoms, aoth):
    fn = pl.kernel(
        _k3_body,
        out_type=(jax.ShapeDtypeStruct((_R2E, _H), jnp.int32),
                  jax.ShapeDtypeStruct((_R2E, _H), jnp.int32)),
        mesh=_sc_mesh(),
        scratch_types=[
            pltpu.VMEM((_CH,), jnp.int32),
            pltpu.VMEM((_CH,), jnp.int32),
            pltpu.VMEM((_CH, _H), jnp.int32),
            pltpu.VMEM((_CH, _H), jnp.int32),
            pltpu.SemaphoreType.DMA,
            pltpu.SemaphoreType.DMA,
        ],
    )
    return fn(ya, yb, atoms, aoth)


def _dot(a, b):
    return lax.dot_general(a, b, (((1,), (0,)), ((), ())),
                           preferred_element_type=_f32,
                           precision=lax.Precision.HIGHEST)


def _wsplit(w):
    return [w[i * _H:(i + 1) * _H] for i in range(7)]


def _pack_bf16_pair(lo_f, hi_f):
    def rnd(x):
        w = lax.bitcast_convert_type(x, jnp.int32)
        return (w + 0x7FFF + ((w >> 16) & 1)) >> 16
    return (rnd(hi_f) << 16) | (rnd(lo_f) & 0xFFFF)


def _feat_body(se_ref, t_ref, u_ref, v_ref, nr_ref, d0_ref, d1_ref, ew1_ref,
               ya_ref, yb_ref, xn_ref):
    w = _wsplit(ew1_ref[...])
    b_m = w[1] + w[2] + w[3] + w[4]
    c_m = w[1] + w[3]
    d_m = w[1] + w[2]
    f_m = w[5] + w[6]
    g_m = w[5]
    se = se_ref[...]
    t = t_ref[...]
    nr = nr_ref[...]
    d = d0_ref[...][:, 0:1] + d1_ref[...][:, 0:1]
    d2 = d * d
    ya = _dot(d * se, b_m) + _dot(t, c_m) + _dot(d * nr, f_m)
    yb = _dot(se, d_m) + _dot(nr, g_m)
    ya_ref[...] = _pack_bf16_pair(ya[:, :_H], ya[:, _H:])
    yb_ref[...] = _pack_bf16_pair(yb[:, :_H], yb[:, _H:])
    xn_ref[...] = jnp.concatenate(
        [nr, d2 * se, d * t, u_ref[...], se, d2 * nr, v_ref[...]], axis=1)


def _run_feat(se, t, u, v, nr, d0, d1, ew1):
    nb, blk = 10, _N // 10
    grid = (nb,)
    io128 = pl.BlockSpec((blk, _H), lambda i: (i, 0))
    return pl.pallas_call(
        _feat_body,
        grid=grid,
        in_specs=[io128, io128, io128, io128, io128, io128, io128,
                  pl.BlockSpec((7 * _H, 2 * _H), lambda i: (0, 0))],
        out_specs=[pl.BlockSpec((blk, _H), lambda i: (i, 0)),
                   pl.BlockSpec((blk, _H), lambda i: (i, 0)),
                   pl.BlockSpec((blk, 7 * _H), lambda i: (i, 0))],
        out_shape=[jax.ShapeDtypeStruct((_N, _H), jnp.int32),
                   jax.ShapeDtypeStruct((_N, _H), jnp.int32),
                   jax.ShapeDtypeStruct((_N, 7 * _H), _f32)],
    )(se, t, u, v, nr, d0, d1, ew1)


def _bn_coeffs(stats_ref, g_ref, b_ref, n):
    sums = stats_ref[0:1, :]
    sqs = stats_ref[1:2, :]
    mean = sums * (1.0 / n)
    var = sqs * (1.0 / n) - mean * mean
    scale = g_ref[...] * lax.rsqrt(var + _EPS)
    shift = b_ref[...] - mean * scale
    return scale, shift


def _unpack_g(g_ref):
    w = g_ref[...]
    lo = lax.bitcast_convert_type(w << 16, _f32)
    hi = lax.bitcast_convert_type((w >> 16) << 16, _f32)
    return jnp.concatenate([lo, hi], axis=1)


def _ep1_body(er_ref, ga_ref, gb_ref, ew1_ref, stats_ref, acc_ref):
    i = pl.program_id(0)

    @pl.when(i == 0)
    def _():
        acc_ref[...] = jnp.zeros_like(acc_ref)

    w = ew1_ref[...]
    a_m = w[0:_H] + w[_H:2 * _H]
    h = _dot(er_ref[...], a_m) + _unpack_g(ga_ref) + _unpack_g(gb_ref)
    acc_ref[0:1, :] += jnp.sum(h, axis=0, keepdims=True)
    acc_ref[1:2, :] += jnp.sum(h * h, axis=0, keepdims=True)
    stats_ref[...] = acc_ref[...]


def _run_ep1(er, ga, gb, ew1, blk=2000):
    nb = _R2E // blk
    return pl.pallas_call(
        _ep1_body,
        grid=(nb,),
        in_specs=[pl.BlockSpec((blk, _H), lambda i: (i, 0)),
                  pl.BlockSpec((blk, _H), lambda i: (i, 0)),
                  pl.BlockSpec((blk, _H), lambda i: (i, 0)),
                  pl.BlockSpec((7 * _H, 2 * _H), lambda i: (0, 0))],
        out_specs=pl.BlockSpec((8, 2 * _H), lambda i: (0, 0)),
        out_shape=jax.ShapeDtypeStruct((8, 2 * _H), _f32),
        scratch_shapes=[pltpu.VMEM((8, 2 * _H), _f32)],
    )(er, ga, gb, ew1)


def _ep2_body(er_ref, ga_ref, gb_ref, ew1_ref, st1_ref, g1_ref, b1_ref,
              w2_ref, h2_ref, stats_ref, acc_ref):
    i = pl.program_id(0)

    @pl.when(i == 0)
    def _():
        acc_ref[...] = jnp.zeros_like(acc_ref)

    w = ew1_ref[...]
    a_m = w[0:_H] + w[_H:2 * _H]
    h = _dot(er_ref[...], a_m) + _unpack_g(ga_ref) + _unpack_g(gb_ref)
    scale, shift = _bn_coeffs(st1_ref, g1_ref, b1_ref, float(_R2E))
    r = jnp.maximum(h * scale + shift, 0.0)
    h2 = _dot(r, w2_ref[...])
    acc_ref[0:1, :] += jnp.sum(h2, axis=0, keepdims=True)
    acc_ref[1:2, :] += jnp.sum(h2 * h2, axis=0, keepdims=True)
    h2_ref[...] = h2
    stats_ref[...] = acc_ref[...]


def _run_ep2(er, ga, gb, ew1, st1, g1, b1, w2, blk=2000):
    nb = _R2E // blk
    return pl.pallas_call(
        _ep2_body,
        grid=(nb,),
        in_specs=[pl.BlockSpec((blk, _H), lambda i: (i, 0)),
                  pl.BlockSpec((blk, _H), lambda i: (i, 0)),
                  pl.BlockSpec((blk, _H), lambda i: (i, 0)),
                  pl.BlockSpec((7 * _H, 2 * _H), lambda i: (0, 0)),
                  pl.BlockSpec((8, 2 * _H), lambda i: (0, 0)),
                  pl.BlockSpec((1, 2 * _H), lambda i: (0, 0)),
                  pl.BlockSpec((1, 2 * _H), lambda i: (0, 0)),
                  pl.BlockSpec((2 * _H, _H), lambda i: (0, 0))],
        out_specs=[pl.BlockSpec((blk, _H), lambda i: (i, 0)),
                   pl.BlockSpec((8, _H), lambda i: (0, 0))],
        out_shape=[jax.ShapeDtypeStruct((_R2E, _H), _f32),
                   jax.ShapeDtypeStruct((8, _H), _f32)],
        scratch_shapes=[pltpu.VMEM((8, _H), _f32)],
    )(er, ga, gb, ew1, st1, g1, b1, w2)


def _run_p3(h2, st2, g2, b2, n_rows, blk):
    nb = h2.shape[0] // blk
    n = float(n_rows)

    def _p3_body(h2_ref, st2_ref, g2_ref, b2_ref, out_ref):
        scale, shift = _bn_coeffs(st2_ref, g2_ref, b2_ref, n)
        out_ref[...] = jnp.maximum(h2_ref[...] * scale + shift, 0.0)

    return pl.pallas_call(
        _p3_body,
        grid=(nb,),
        in_specs=[pl.BlockSpec((blk, _H), lambda i: (i, 0)),
                  pl.BlockSpec((8, _H), lambda i: (0, 0)),
                  pl.BlockSpec((1, _H), lambda i: (0, 0)),
                  pl.BlockSpec((1, _H), lambda i: (0, 0))],
        out_specs=pl.BlockSpec((blk, _H), lambda i: (i, 0)),
        out_shape=jax.ShapeDtypeStruct((h2.shape[0], _H), _f32),
    )(h2, st2, g2, b2)


def _np1_body(xn_ref, nw1_ref, h1_ref, stats_ref, acc_ref):
    i = pl.program_id(0)

    @pl.when(i == 0)
    def _():
        acc_ref[...] = jnp.zeros_like(acc_ref)

    w = _wsplit(nw1_ref[...])
    wcat = jnp.concatenate(
        [w[0], w[1] + w[2] + w[3] + w[4], w[1] + w[3], w[1] + w[2],
         w[1], w[5] + w[6], w[5]], axis=0)
    h = _dot(xn_ref[...], wcat)
    acc_ref[0:1, :] += jnp.sum(h, axis=0, keepdims=True)
    acc_ref[1:2, :] += jnp.sum(h * h, axis=0, keepdims=True)
    h1_ref[...] = h
    stats_ref[...] = acc_ref[...]


def _run_np1(xn, nw1):
    nb, blk = 10, _N // 10
    return pl.pallas_call(
        _np1_body,
        grid=(nb,),
        in_specs=[pl.BlockSpec((blk, 7 * _H), lambda i: (i, 0)),
                  pl.BlockSpec((7 * _H, 2 * _H), lambda i: (0, 0))],
        out_specs=[pl.BlockSpec((blk, 2 * _H), lambda i: (i, 0)),
                   pl.BlockSpec((8, 2 * _H), lambda i: (0, 0))],
        out_shape=[jax.ShapeDtypeStruct((_N, 2 * _H), _f32),
                   jax.ShapeDtypeStruct((8, 2 * _H), _f32)],
        scratch_shapes=[pltpu.VMEM((8, 2 * _H), _f32)],
    )(xn, nw1)


def _np2_body(h1_ref, st1_ref, g1_ref, b1_ref, w2_ref,
              h2_ref, stats_ref, acc_ref):
    i = pl.program_id(0)

    @pl.when(i == 0)
    def _():
        acc_ref[...] = jnp.zeros_like(acc_ref)

    scale, shift = _bn_coeffs(st1_ref, g1_ref, b1_ref, float(_N))
    r = jnp.maximum(h1_ref[...] * scale + shift, 0.0)
    h2 = _dot(r, w2_ref[...])
    acc_ref[0:1, :] += jnp.sum(h2, axis=0, keepdims=True)
    acc_ref[1:2, :] += jnp.sum(h2 * h2, axis=0, keepdims=True)
    h2_ref[...] = h2
    stats_ref[...] = acc_ref[...]


def _run_np2(h1, st1, g1, b1, w2):
    nb, blk = 10, _N // 10
    return pl.pallas_call(
        _np2_body,
        grid=(nb,),
        in_specs=[pl.BlockSpec((blk, 2 * _H), lambda i: (i, 0)),
                  pl.BlockSpec((8, 2 * _H), lambda i: (0, 0)),
                  pl.BlockSpec((1, 2 * _H), lambda i: (0, 0)),
                  pl.BlockSpec((1, 2 * _H), lambda i: (0, 0)),
                  pl.BlockSpec((2 * _H, _H), lambda i: (0, 0))],
        out_specs=[pl.BlockSpec((blk, _H), lambda i: (i, 0)),
                   pl.BlockSpec((8, _H), lambda i: (0, 0))],
        out_shape=[jax.ShapeDtypeStruct((_N, _H), _f32),
                   jax.ShapeDtypeStruct((8, _H), _f32)],
        scratch_shapes=[pltpu.VMEM((8, _H), _f32)],
    )(h1, st1, g1, b1, w2)


def kernel(node_rep, edge_rep, edge_index,
           node_W1, node_g1, node_b1, node_W2, node_g2, node_b2,
           edge_W1, edge_g1, edge_b1, edge_W2, edge_g2, edge_b2):
    u, v = edge_index[0], edge_index[1]
    atoms = jnp.stack([u, v], axis=1).reshape(-1).astype(jnp.int32)
    aoth = jnp.stack([v, u], axis=1).reshape(-1).astype(jnp.int32)

    idx2 = jnp.concatenate([atoms, aoth])
    se, t = _run_k1(edge_rep, idx2)
    d0, d1 = _run_kd(atoms)
    tab2 = jnp.concatenate(
        [se, node_rep, jnp.zeros((_NP - _N, _H), _f32)], axis=0)
    uu, vv = _run_k2(tab2, atoms, aoth)
    ya, yb, xn = _run_feat(se[:_N], t[:_N], uu[:_N], vv[:_N],
                           node_rep, d0[:_N], d1[:_N], edge_W1)
    ga, gb = _run_k3(ya, yb, atoms, aoth)

    eg1 = edge_g1.reshape(1, -1)
    eb1 = edge_b1.reshape(1, -1)
    st1 = _run_ep1(edge_rep, ga, gb, edge_W1)
    h2e, st2 = _run_ep2(edge_rep, ga, gb, edge_W1, st1, eg1, eb1, edge_W2)
    edge_out = _run_p3(h2e, st2, edge_g2.reshape(1, -1),
                       edge_b2.reshape(1, -1), _R2E, 2000)

    nst1_h1 = _run_np1(xn, node_W1)
    h1n, nst1 = nst1_h1
    h2n, nst2 = _run_np2(h1n, nst1, node_g1.reshape(1, -1),
                         node_b1.reshape(1, -1), node_W2)
    node_out = _run_p3(h2n, nst2, node_g2.reshape(1, -1),
                       node_b2.reshape(1, -1), _N, 1000)

    return (node_out, edge_out)


# speedup vs baseline: 3.4237x; 1.2491x over previous
import functools

import jax
import jax.numpy as jnp
from jax import lax
from jax.experimental import pallas as pl
from jax.experimental.pallas import tpu as pltpu
from jax.experimental.pallas import tpu_sc as plsc

_N = 10000
_E = 160000
_H = 128
_R2E = 2 * _E
_CH = 80
_NT = 16
_NP = 10240
_NPT = _NP // _NT
_EPS = 1e-5

_f32 = jnp.float32


def _sc_mesh():
    return plsc.VectorSubcoreMesh(core_axis_name="c", subcore_axis_name="s")


def _k1_body(era, idx2_h, zz,
             se_o, t_o,
             idx0_v, idx1_v, rows0_v, rows1_v, tab_sh,
             si0, si1, sr0, sr1):
    c = lax.axis_index("c")
    s = lax.axis_index("s")
    pltpu.sync_copy(zz, tab_sh.at[pl.ds(s * _NPT, _NPT)])
    plsc.subcore_barrier()

    rpt = _R2E // _NT
    nch = rpt // _CH
    ibase = c * _R2E
    bufs = ((idx0_v, rows0_v, si0, sr0), (idx1_v, rows1_v, si1, sr1))

    def issue(jj, b):
        idx_v, rows_v, si, sr = bufs[b]
        r0 = s * rpt + jj * _CH
        pltpu.async_copy(idx2_h.at[pl.ds(ibase + r0, _CH)], idx_v, si)
        pltpu.async_copy(era.at[pl.ds(r0, _CH)], rows_v, sr)

    def wait(jj, b):
        idx_v, rows_v, si, sr = bufs[b]
        r0 = s * rpt + jj * _CH
        pltpu.make_async_copy(idx2_h.at[pl.ds(ibase + r0, _CH)],
                              idx_v, si).wait()
        pltpu.make_async_copy(era.at[pl.ds(r0, _CH)], rows_v, sr).wait()

    issue(0, 0)

    def outer(j2, carry):
        for b in range(2):
            jj = 2 * j2 + b

            @pl.when(jj + 1 < nch)
            def _():
                issue(jj + 1, 1 - b)

            wait(jj, b)
            idx_v, rows_v, _, _ = bufs[b]
            pltpu.sync_copy(rows_v, tab_sh.at[idx_v], add=True)
        return carry

    lax.fori_loop(0, nch // 2, outer, 0)
    plsc.subcore_barrier()

    @pl.when(c == 0)
    def _():
        pltpu.sync_copy(tab_sh.at[pl.ds(s * _NPT, _NPT)],
                        se_o.at[pl.ds(s * _NPT, _NPT)])

    @pl.when(c == 1)
    def _():
        pltpu.sync_copy(tab_sh.at[pl.ds(s * _NPT, _NPT)],
                        t_o.at[pl.ds(s * _NPT, _NPT)])


def _run_k1(edge_rep, idx2):
    zz = jnp.zeros((_NPT, _H), _f32)
    fn = pl.kernel(
        _k1_body,
        out_type=(jax.ShapeDtypeStruct((_NP, _H), _f32),
                  jax.ShapeDtypeStruct((_NP, _H), _f32)),
        mesh=_sc_mesh(),
        scratch_types=[
            pltpu.VMEM((_CH,), jnp.int32),
            pltpu.VMEM((_CH,), jnp.int32),
            pltpu.VMEM((_CH, _H), _f32),
            pltpu.VMEM((_CH, _H), _f32),
            pltpu.VMEM_SHARED((_NP, _H), _f32),
            pltpu.SemaphoreType.DMA,
            pltpu.SemaphoreType.DMA,
            pltpu.SemaphoreType.DMA,
            pltpu.SemaphoreType.DMA,
        ],
    )
    return fn(edge_rep, idx2, zz)


def _kd_body(atoms_h, zz, oo,
             d0_o, d1_o,
             idx0_v, idx1_v, ones_v, tab_sh, si0, si1):
    c = lax.axis_index("c")
    s = lax.axis_index("s")
    pltpu.sync_copy(zz, tab_sh.at[pl.ds(s * _NPT, _NPT)])
    pltpu.sync_copy(oo, ones_v)
    plsc.subcore_barrier()

    rpc = _R2E // 2
    rpt = rpc // _NT
    nch = rpt // _CH
    bufs = ((idx0_v, si0), (idx1_v, si1))

    def issue(jj, b):
        idx_v, si = bufs[b]
        r0 = c * rpc + s * rpt + jj * _CH
        pltpu.async_copy(atoms_h.at[pl.ds(r0, _CH)], idx_v, si)

    def wait(jj, b):
        idx_v, si = bufs[b]
        r0 = c * rpc + s * rpt + jj * _CH
        pltpu.make_async_copy(atoms_h.at[pl.ds(r0, _CH)], idx_v, si).wait()

    issue(0, 0)

    def outer(j2, carry):
        for b in range(2):
            jj = 2 * j2 + b

            @pl.when(jj + 1 < nch)
            def _():
                issue(jj + 1, 1 - b)

            @pl.when(jj < nch)
            def _():
                wait(jj, b)
                idx_v, _ = bufs[b]
                pltpu.sync_copy(ones_v, tab_sh.at[idx_v], add=True)
        return carry

    lax.fori_loop(0, (nch + 1) // 2, outer, 0)
    plsc.subcore_barrier()

    @pl.when(c == 0)
    def _():
        pltpu.sync_copy(tab_sh.at[pl.ds(s * _NPT, _NPT)],
                        d0_o.at[pl.ds(s * _NPT, _NPT)])

    @pl.when(c == 1)
    def _():
        pltpu.sync_copy(tab_sh.at[pl.ds(s * _NPT, _NPT)],
                        d1_o.at[pl.ds(s * _NPT, _NPT)])


def _run_kd(atoms):
    zz = jnp.zeros((_NPT, _H), _f32)
    oo = jnp.ones((_CH, _H), _f32)
    fn = pl.kernel(
        _kd_body,
        out_type=(jax.ShapeDtypeStruct((_NP, _H), _f32),
                  jax.ShapeDtypeStruct((_NP, _H), _f32)),
        mesh=_sc_mesh(),
        scratch_types=[
            pltpu.VMEM((_CH,), jnp.int32),
            pltpu.VMEM((_CH,), jnp.int32),
            pltpu.VMEM((_CH, _H), _f32),
            pltpu.VMEM_SHARED((_NP, _H), _f32),
            pltpu.SemaphoreType.DMA,
            pltpu.SemaphoreType.DMA,
        ],
    )
    return fn(atoms, zz, oo)


def _k2_body(tab2_h, atoms_h, aoth_h, zz,
             u_o, v_o,
             idxd0, idxs0, idxd1, idxs1, rows0_v, rows1_v, tab_sh,
             sl0, sl1, sg0, sg1):
    c = lax.axis_index("c")
    s = lax.axis_index("s")
    pltpu.sync_copy(zz, tab_sh.at[pl.ds(s * _NPT, _NPT)])
    plsc.subcore_barrier()

    rpt = _R2E // _NT
    nch = rpt // _CH
    off = c * _NP
    bufs = ((idxd0, idxs0, rows0_v, sl0, sg0),
            (idxd1, idxs1, rows1_v, sl1, sg1))

    def issue(jj, b):
        idxd, idxs, _, sl, _ = bufs[b]
        r0 = s * rpt + jj * _CH
        pltpu.async_copy(atoms_h.at[pl.ds(r0, _CH)], idxd, sl)
        pltpu.async_copy(aoth_h.at[pl.ds(r0, _CH)], idxs, sl)

    def wait_l(jj, b):
        idxd, idxs, _, sl, _ = bufs[b]
        r0 = s * rpt + jj * _CH
        pltpu.make_async_copy(atoms_h.at[pl.ds(r0, _CH)], idxd, sl).wait()
        pltpu.make_async_copy(aoth_h.at[pl.ds(r0, _CH)], idxs, sl).wait()

    issue(0, 0)

    def outer(j2, carry):
        for b in range(2):
            jj = 2 * j2 + b

            @pl.when(jj + 1 < nch)
            def _():
                issue(jj + 1, 1 - b)

            wait_l(jj, b)
            idxd, idxs, rows_v, _, sg = bufs[b]
            for k in range(_CH // 16):
                sl_ = pl.ds(k * 16, 16)
                idxs[sl_] = idxs[sl_] + off
            pltpu.async_copy(tab2_h.at[idxs], rows_v, sg).wait()
            pltpu.sync_copy(rows_v, tab_sh.at[idxd], add=True)
        return carry

    lax.fori_loop(0, nch // 2, outer, 0)
    plsc.subcore_barrier()

    @pl.when(c == 0)
    def _():
        pltpu.sync_copy(tab_sh.at[pl.ds(s * _NPT, _NPT)],
                        u_o.at[pl.ds(s * _NPT, _NPT)])

    @pl.when(c == 1)
    def _():
        pltpu.sync_copy(tab_sh.at[pl.ds(s * _NPT, _NPT)],
                        v_o.at[pl.ds(s * _NPT, _NPT)])


def _run_k2(tab2, atoms, aoth):
    zz = jnp.zeros((_NPT, _H), _f32)
    fn = pl.kernel(
        _k2_body,
        out_type=(jax.ShapeDtypeStruct((_NP, _H), _f32),
                  jax.ShapeDtypeStruct((_NP, _H), _f32)),
        mesh=_sc_mesh(),
        scratch_types=[
            pltpu.VMEM((_CH,), jnp.int32),
            pltpu.VMEM((_CH,), jnp.int32),
            pltpu.VMEM((_CH,), jnp.int32),
            pltpu.VMEM((_CH,), jnp.int32),
            pltpu.VMEM((_CH, _H), _f32),
            pltpu.VMEM((_CH, _H), _f32),
            pltpu.VMEM_SHARED((_NP, _H), _f32),
            pltpu.SemaphoreType.DMA,
            pltpu.SemaphoreType.DMA,
            pltpu.SemaphoreType.DMA,
            pltpu.SemaphoreType.DMA,
        ],
    )
    return fn(tab2, atoms, aoth, zz)


_bf16 = jnp.bfloat16


def _k3_body(ya_h, yb_h, atoms_h, aoth_h, ga_o, gb_o,
             idxa0, idxb0, idxa1, idxb1, bufa0, bufb0, bufa1, bufb1,
             sl0, sl1, sg0, sg1, sw0, sw1):
    c = lax.axis_index("c")
    s = lax.axis_index("s")
    wid = s * 2 + c
    rpw = _R2E // 32
    nch = rpw // _CH
    bufs = ((idxa0, idxb0, bufa0, bufb0, sl0, sg0, sw0),
            (idxa1, idxb1, bufa1, bufb1, sl1, sg1, sw1))

    def issue_l(jj, b):
        idxa, idxb, _, _, sl, _, _ = bufs[b]
        r0 = wid * rpw + jj * _CH
        pltpu.async_copy(atoms_h.at[pl.ds(r0, _CH)], idxa, sl)
        pltpu.async_copy(aoth_h.at[pl.ds(r0, _CH)], idxb, sl)

    def wait_l(jj, b):
        idxa, idxb, _, _, sl, _, _ = bufs[b]
        r0 = wid * rpw + jj * _CH
        pltpu.make_async_copy(atoms_h.at[pl.ds(r0, _CH)], idxa, sl).wait()
        pltpu.make_async_copy(aoth_h.at[pl.ds(r0, _CH)], idxb, sl).wait()

    def wait_w(jj, b):
        _, _, bufa, bufb, _, _, sw = bufs[b]
        r0 = wid * rpw + jj * _CH
        pltpu.make_async_copy(bufa, ga_o.at[pl.ds(r0, _CH)], sw).wait()
        pltpu.make_async_copy(bufb, gb_o.at[pl.ds(r0, _CH)], sw).wait()

    def chunk(jj, b):
        idxa, idxb, bufa, bufb, _, sg, sw = bufs[b]
        r0 = wid * rpw + jj * _CH

        @pl.when(jj + 1 < nch)
        def _():
            issue_l(jj + 1, 1 - b)

        wait_l(jj, b)

        @pl.when(jj >= 2)
        def _():
            wait_w(jj - 2, b)

        ca = pltpu.async_copy(ya_h.at[idxa], bufa, sg)
        cb = pltpu.async_copy(yb_h.at[idxb], bufb, sg)
        ca.wait()
        cb.wait()
        pltpu.async_copy(bufa, ga_o.at[pl.ds(r0, _CH)], sw)
        pltpu.async_copy(bufb, gb_o.at[pl.ds(r0, _CH)], sw)

    issue_l(0, 0)

    def outer(j2, carry):
        for b in range(2):
            jj = 2 * j2 + b

            @pl.when(jj < nch)
            def _():
                chunk(jj, b)
        return carry

    lax.fori_loop(0, (nch + 1) // 2, outer, 0)
    wait_w(nch - 2, (nch - 2) % 2)
    wait_w(nch - 1, (nch - 1) % 2)


def _run_k3(ya, yb, atoms, aoth):
    fn = pl.kernel(
        _k3_body,
        out_type=(jax.ShapeDtypeStruct((_R2E, _H), jnp.int32),
                  jax.ShapeDtypeStruct((_R2E, _H), jnp.int32)),
        mesh=_sc_mesh(),
        scratch_types=[
            pltpu.VMEM((_CH,), jnp.int32),
            pltpu.VMEM((_CH,), jnp.int32),
            pltpu.VMEM((_CH,), jnp.int32),
            pltpu.VMEM((_CH,), jnp.int32),
            pltpu.VMEM((_CH, _H), jnp.int32),
            pltpu.VMEM((_CH, _H), jnp.int32),
            pltpu.VMEM((_CH, _H), jnp.int32),
            pltpu.VMEM((_CH, _H), jnp.int32),
            pltpu.SemaphoreType.DMA,
            pltpu.SemaphoreType.DMA,
            pltpu.SemaphoreType.DMA,
            pltpu.SemaphoreType.DMA,
            pltpu.SemaphoreType.DMA,
            pltpu.SemaphoreType.DMA,
        ],
    )
    return fn(ya, yb, atoms, aoth)


def _dot(a, b):
    return lax.dot_general(a, b, (((1,), (0,)), ((), ())),
                           preferred_element_type=_f32,
                           precision=lax.Precision.HIGHEST)


def _wsplit(w):
    return [w[i * _H:(i + 1) * _H] for i in range(7)]


def _pack_bf16_pair(lo_f, hi_f):
    def rnd(x):
        w = lax.bitcast_convert_type(x, jnp.int32)
        return (w + 0x7FFF + ((w >> 16) & 1)) >> 16
    return (rnd(hi_f) << 16) | (rnd(lo_f) & 0xFFFF)


def _feat_body(se_ref, t_ref, u_ref, v_ref, nr_ref, d0_ref, d1_ref, ew1_ref,
               ya_ref, yb_ref, xn_ref):
    w = _wsplit(ew1_ref[...])
    b_m = w[1] + w[2] + w[3] + w[4]
    c_m = w[1] + w[3]
    d_m = w[1] + w[2]
    f_m = w[5] + w[6]
    g_m = w[5]
    se = se_ref[...]
    t = t_ref[...]
    nr = nr_ref[...]
    d = d0_ref[...][:, 0:1] + d1_ref[...][:, 0:1]
    d2 = d * d
    ya = _dot(d * se, b_m) + _dot(t, c_m) + _dot(d * nr, f_m)
    yb = _dot(se, d_m) + _dot(nr, g_m)
    ya_ref[...] = _pack_bf16_pair(ya[:, :_H], ya[:, _H:])
    yb_ref[...] = _pack_bf16_pair(yb[:, :_H], yb[:, _H:])
    xn_ref[...] = jnp.concatenate(
        [nr, d2 * se, d * t, u_ref[...], se, d2 * nr, v_ref[...]], axis=1)


def _run_feat(se, t, u, v, nr, d0, d1, ew1):
    nb, blk = 10, _N // 10
    grid = (nb,)
    io128 = pl.BlockSpec((blk, _H), lambda i: (i, 0))
    return pl.pallas_call(
        _feat_body,
        grid=grid,
        in_specs=[io128, io128, io128, io128, io128, io128, io128,
                  pl.BlockSpec((7 * _H, 2 * _H), lambda i: (0, 0))],
        out_specs=[pl.BlockSpec((blk, _H), lambda i: (i, 0)),
                   pl.BlockSpec((blk, _H), lambda i: (i, 0)),
                   pl.BlockSpec((blk, 7 * _H), lambda i: (i, 0))],
        out_shape=[jax.ShapeDtypeStruct((_N, _H), jnp.int32),
                   jax.ShapeDtypeStruct((_N, _H), jnp.int32),
                   jax.ShapeDtypeStruct((_N, 7 * _H), _f32)],
    )(se, t, u, v, nr, d0, d1, ew1)


def _bn_coeffs(stats_ref, g_ref, b_ref, n):
    sums = stats_ref[0:1, :]
    sqs = stats_ref[1:2, :]
    mean = sums * (1.0 / n)
    var = sqs * (1.0 / n) - mean * mean
    scale = g_ref[...] * lax.rsqrt(var + _EPS)
    shift = b_ref[...] - mean * scale
    return scale, shift


def _unpack_g(g_ref):
    w = g_ref[...]
    lo = lax.bitcast_convert_type(w << 16, _f32)
    hi = lax.bitcast_convert_type((w >> 16) << 16, _f32)
    return jnp.concatenate([lo, hi], axis=1)


def _ep1_body(er_ref, ga_ref, gb_ref, ew1_ref, stats_ref, acc_ref):
    i = pl.program_id(0)

    @pl.when(i == 0)
    def _():
        acc_ref[...] = jnp.zeros_like(acc_ref)

    w = ew1_ref[...]
    a_m = w[0:_H] + w[_H:2 * _H]
    h = _dot(er_ref[...], a_m) + _unpack_g(ga_ref) + _unpack_g(gb_ref)
    acc_ref[0:1, :] += jnp.sum(h, axis=0, keepdims=True)
    acc_ref[1:2, :] += jnp.sum(h * h, axis=0, keepdims=True)
    stats_ref[...] = acc_ref[...]


def _run_ep1(er, ga, gb, ew1, blk=2000):
    nb = _R2E // blk
    return pl.pallas_call(
        _ep1_body,
        grid=(nb,),
        in_specs=[pl.BlockSpec((blk, _H), lambda i: (i, 0)),
                  pl.BlockSpec((blk, _H), lambda i: (i, 0)),
                  pl.BlockSpec((blk, _H), lambda i: (i, 0)),
                  pl.BlockSpec((7 * _H, 2 * _H), lambda i: (0, 0))],
        out_specs=pl.BlockSpec((8, 2 * _H), lambda i: (0, 0)),
        out_shape=jax.ShapeDtypeStruct((8, 2 * _H), _f32),
        scratch_shapes=[pltpu.VMEM((8, 2 * _H), _f32)],
    )(er, ga, gb, ew1)


def _ep2_body(er_ref, ga_ref, gb_ref, ew1_ref, st1_ref, g1_ref, b1_ref,
              w2_ref, h2_ref, stats_ref, acc_ref):
    i = pl.program_id(0)

    @pl.when(i == 0)
    def _():
        acc_ref[...] = jnp.zeros_like(acc_ref)

    w = ew1_ref[...]
    a_m = w[0:_H] + w[_H:2 * _H]
    h = _dot(er_ref[...], a_m) + _unpack_g(ga_ref) + _unpack_g(gb_ref)
    scale, shift = _bn_coeffs(st1_ref, g1_ref, b1_ref, float(_R2E))
    r = jnp.maximum(h * scale + shift, 0.0)
    h2 = _dot(r, w2_ref[...])
    acc_ref[0:1, :] += jnp.sum(h2, axis=0, keepdims=True)
    acc_ref[1:2, :] += jnp.sum(h2 * h2, axis=0, keepdims=True)
    h2_ref[...] = h2
    stats_ref[...] = acc_ref[...]


def _run_ep2(er, ga, gb, ew1, st1, g1, b1, w2, blk=2000):
    nb = _R2E // blk
    return pl.pallas_call(
        _ep2_body,
        grid=(nb,),
        in_specs=[pl.BlockSpec((blk, _H), lambda i: (i, 0)),
                  pl.BlockSpec((blk, _H), lambda i: (i, 0)),
                  pl.BlockSpec((blk, _H), lambda i: (i, 0)),
                  pl.BlockSpec((7 * _H, 2 * _H), lambda i: (0, 0)),
                  pl.BlockSpec((8, 2 * _H), lambda i: (0, 0)),
                  pl.BlockSpec((1, 2 * _H), lambda i: (0, 0)),
                  pl.BlockSpec((1, 2 * _H), lambda i: (0, 0)),
                  pl.BlockSpec((2 * _H, _H), lambda i: (0, 0))],
        out_specs=[pl.BlockSpec((blk, _H), lambda i: (i, 0)),
                   pl.BlockSpec((8, _H), lambda i: (0, 0))],
        out_shape=[jax.ShapeDtypeStruct((_R2E, _H), _f32),
                   jax.ShapeDtypeStruct((8, _H), _f32)],
        scratch_shapes=[pltpu.VMEM((8, _H), _f32)],
    )(er, ga, gb, ew1, st1, g1, b1, w2)


def _run_p3(h2, st2, g2, b2, n_rows, blk):
    nb = h2.shape[0] // blk
    n = float(n_rows)

    def _p3_body(h2_ref, st2_ref, g2_ref, b2_ref, out_ref):
        scale, shift = _bn_coeffs(st2_ref, g2_ref, b2_ref, n)
        out_ref[...] = jnp.maximum(h2_ref[...] * scale + shift, 0.0)

    return pl.pallas_call(
        _p3_body,
        grid=(nb,),
        in_specs=[pl.BlockSpec((blk, _H), lambda i: (i, 0)),
                  pl.BlockSpec((8, _H), lambda i: (0, 0)),
                  pl.BlockSpec((1, _H), lambda i: (0, 0)),
                  pl.BlockSpec((1, _H), lambda i: (0, 0))],
        out_specs=pl.BlockSpec((blk, _H), lambda i: (i, 0)),
        out_shape=jax.ShapeDtypeStruct((h2.shape[0], _H), _f32),
    )(h2, st2, g2, b2)


def _np1_body(xn_ref, nw1_ref, h1_ref, stats_ref, acc_ref):
    i = pl.program_id(0)

    @pl.when(i == 0)
    def _():
        acc_ref[...] = jnp.zeros_like(acc_ref)

    w = _wsplit(nw1_ref[...])
    wcat = jnp.concatenate(
        [w[0], w[1] + w[2] + w[3] + w[4], w[1] + w[3], w[1] + w[2],
         w[1], w[5] + w[6], w[5]], axis=0)
    h = _dot(xn_ref[...], wcat)
    acc_ref[0:1, :] += jnp.sum(h, axis=0, keepdims=True)
    acc_ref[1:2, :] += jnp.sum(h * h, axis=0, keepdims=True)
    h1_ref[...] = h
    stats_ref[...] = acc_ref[...]


def _run_np1(xn, nw1):
    nb, blk = 10, _N // 10
    return pl.pallas_call(
        _np1_body,
        grid=(nb,),
        in_specs=[pl.BlockSpec((blk, 7 * _H), lambda i: (i, 0)),
                  pl.BlockSpec((7 * _H, 2 * _H), lambda i: (0, 0))],
        out_specs=[pl.BlockSpec((blk, 2 * _H), lambda i: (i, 0)),
                   pl.BlockSpec((8, 2 * _H), lambda i: (0, 0))],
        out_shape=[jax.ShapeDtypeStruct((_N, 2 * _H), _f32),
                   jax.ShapeDtypeStruct((8, 2 * _H), _f32)],
        scratch_shapes=[pltpu.VMEM((8, 2 * _H), _f32)],
    )(xn, nw1)


def _np2_body(h1_ref, st1_ref, g1_ref, b1_ref, w2_ref,
              h2_ref, stats_ref, acc_ref):
    i = pl.program_id(0)

    @pl.when(i == 0)
    def _():
        acc_ref[...] = jnp.zeros_like(acc_ref)

    scale, shift = _bn_coeffs(st1_ref, g1_ref, b1_ref, float(_N))
    r = jnp.maximum(h1_ref[...] * scale + shift, 0.0)
    h2 = _dot(r, w2_ref[...])
    acc_ref[0:1, :] += jnp.sum(h2, axis=0, keepdims=True)
    acc_ref[1:2, :] += jnp.sum(h2 * h2, axis=0, keepdims=True)
    h2_ref[...] = h2
    stats_ref[...] = acc_ref[...]


def _run_np2(h1, st1, g1, b1, w2):
    nb, blk = 10, _N // 10
    return pl.pallas_call(
        _np2_body,
        grid=(nb,),
        in_specs=[pl.BlockSpec((blk, 2 * _H), lambda i: (i, 0)),
                  pl.BlockSpec((8, 2 * _H), lambda i: (0, 0)),
                  pl.BlockSpec((1, 2 * _H), lambda i: (0, 0)),
                  pl.BlockSpec((1, 2 * _H), lambda i: (0, 0)),
                  pl.BlockSpec((2 * _H, _H), lambda i: (0, 0))],
        out_specs=[pl.BlockSpec((blk, _H), lambda i: (i, 0)),
                   pl.BlockSpec((8, _H), lambda i: (0, 0))],
        out_shape=[jax.ShapeDtypeStruct((_N, _H), _f32),
                   jax.ShapeDtypeStruct((8, _H), _f32)],
        scratch_shapes=[pltpu.VMEM((8, _H), _f32)],
    )(h1, st1, g1, b1, w2)


def kernel(node_rep, edge_rep, edge_index,
           node_W1, node_g1, node_b1, node_W2, node_g2, node_b2,
           edge_W1, edge_g1, edge_b1, edge_W2, edge_g2, edge_b2):
    u, v = edge_index[0], edge_index[1]
    atoms = jnp.stack([u, v], axis=1).reshape(-1).astype(jnp.int32)
    aoth = jnp.stack([v, u], axis=1).reshape(-1).astype(jnp.int32)

    idx2 = jnp.concatenate([atoms, aoth])
    se, t = _run_k1(edge_rep, idx2)
    d0, d1 = _run_kd(atoms)
    tab2 = jnp.concatenate(
        [se, node_rep, jnp.zeros((_NP - _N, _H), _f32)], axis=0)
    uu, vv = _run_k2(tab2, atoms, aoth)
    ya, yb, xn = _run_feat(se[:_N], t[:_N], uu[:_N], vv[:_N],
                           node_rep, d0[:_N], d1[:_N], edge_W1)
    ga, gb = _run_k3(ya, yb, atoms, aoth)

    eg1 = edge_g1.reshape(1, -1)
    eb1 = edge_b1.reshape(1, -1)
    st1 = _run_ep1(edge_rep, ga, gb, edge_W1)
    h2e, st2 = _run_ep2(edge_rep, ga, gb, edge_W1, st1, eg1, eb1, edge_W2)
    edge_out = _run_p3(h2e, st2, edge_g2.reshape(1, -1),
                       edge_b2.reshape(1, -1), _R2E, 2000)

    nst1_h1 = _run_np1(xn, node_W1)
    h1n, nst1 = nst1_h1
    h2n, nst2 = _run_np2(h1n, nst1, node_g1.reshape(1, -1),
                         node_b1.reshape(1, -1), node_W2)
    node_out = _run_p3(h2n, nst2, node_g2.reshape(1, -1),
                       node_b2.reshape(1, -1), _N, 1000)

    return (node_out, edge_out)
